# baseline (speedup 1.0000x reference)
import functools

import jax
import jax.numpy as jnp
import numpy as np
from jax import lax
from jax.experimental import pallas as pl
from jax.experimental.pallas import tpu as pltpu
from jax.experimental.pallas import tpu_sc as plsc

_IDX_CHUNK = 128


def _sc_gather(table, idx):
    B = idx.shape[0]
    D = table.shape[1]
    info = plsc.get_sparse_core_info()
    nw = info.num_cores * info.num_subcores
    b_per_w = B // nw
    n_chunks = b_per_w // _IDX_CHUNK
    mesh = plsc.VectorSubcoreMesh(core_axis_name="c", subcore_axis_name="s")

    @functools.partial(
        pl.kernel,
        mesh=mesh,
        compiler_params=pltpu.CompilerParams(use_tc_tiling_on_sc=False),
        out_type=jax.ShapeDtypeStruct((B, D), jnp.float32),
        scratch_types=[
            pltpu.VMEM((b_per_w,), jnp.int32),
            pltpu.VMEM((b_per_w, D), jnp.float32),
            pltpu.SemaphoreType.DMA,
        ],
    )
    def gather_kernel(table_hbm, idx_hbm, out_hbm, idx_v, rows_v, sem):
        wid = lax.axis_index("s") * info.num_cores + lax.axis_index("c")
        base = wid * b_per_w
        pltpu.sync_copy(idx_hbm.at[pl.ds(base, b_per_w)], idx_v)
        copies = []
        for j in range(n_chunks):
            copies.append(
                pltpu.async_copy(
                    table_hbm.at[idx_v.at[pl.ds(j * _IDX_CHUNK, _IDX_CHUNK)]],
                    rows_v.at[pl.ds(j * _IDX_CHUNK, _IDX_CHUNK)],
                    sem,
                )
            )
        for c in copies:
            c.wait()
        pltpu.sync_copy(rows_v, out_hbm.at[pl.ds(base, b_per_w)])

    return gather_kernel(table, idx.astype(jnp.int32))


_LANCZOS_G = 7.0
_LANCZOS_C = (
    0.99999999999980993,
    676.5203681218851,
    -1259.1392167224028,
    771.32342877765313,
    -176.61502916214059,
    12.507343278686905,
    -0.13857109526572012,
    9.9843695780195716e-6,
    1.5056327351493116e-7,
)
_HALF_LOG_2PI = 0.9189385332046727


def _lgamma(x):
    z = x - 1.0
    a = jnp.full_like(x, _LANCZOS_C[0])
    for i in range(1, 9):
        a = a + _LANCZOS_C[i] / (z + i)
    t = z + _LANCZOS_G + 0.5
    return _HALF_LOG_2PI + (z + 0.5) * jnp.log(t) - t + jnp.log(a)


def _digamma(x):
    shift = jnp.zeros_like(x)
    for k in range(6):
        shift = shift + 1.0 / (x + k)
    y = x + 6.0
    inv = 1.0 / y
    inv2 = inv * inv
    series = (
        jnp.log(y)
        - 0.5 * inv
        - inv2 * (1.0 / 12.0 - inv2 * (1.0 / 120.0 - inv2 / 252.0))
    )
    return series - shift


U32 = jnp.uint32
_R1 = (13, 15, 26, 6)
_R2 = (17, 29, 16, 24)
_C3 = np.uint32(0x1BD11BDA)
_LO = np.nextafter(np.float32(-1.0), np.float32(0.0))
_SQRT2 = np.array(np.sqrt(2), np.float32)
_THIRD = np.float32(1.0 / 3.0)
_SQUEEZE = np.float32(0.0331)
_MK1 = np.uint32(0)
_MK2 = np.uint32(12345)


def _tf_rounds(x0, x1, rots):
    for r in rots:
        x0 = x0 + x1
        x1 = (x1 << U32(r)) | (x1 >> U32(32 - r))
        x1 = x0 ^ x1
    return x0, x1


def _threefry(k, c0, c1):
    k1, k2 = k
    ks2 = k1 ^ k2 ^ _C3
    x0 = c0 + k1
    x1 = c1 + k2
    x0, x1 = _tf_rounds(x0, x1, _R1)
    x0 = x0 + k2
    x1 = x1 + ks2 + U32(1)
    x0, x1 = _tf_rounds(x0, x1, _R2)
    x0 = x0 + ks2
    x1 = x1 + k1 + U32(2)
    x0, x1 = _tf_rounds(x0, x1, _R1)
    x0 = x0 + k1
    x1 = x1 + k2 + U32(3)
    x0, x1 = _tf_rounds(x0, x1, _R2)
    x0 = x0 + k2
    x1 = x1 + ks2 + U32(4)
    x0, x1 = _tf_rounds(x0, x1, _R1)
    x0 = x0 + ks2
    x1 = x1 + k1 + U32(5)
    return x0, x1


def _u01(bits):
    fb = (bits >> U32(9)) | U32(0x3F800000)
    return lax.bitcast_convert_type(fb, jnp.float32) - np.float32(1.0)


def _bits(k):
    z = jnp.zeros_like(k[0])
    y0, y1 = _threefry(k, z, z)
    return y0 ^ y1


def _erf_inv(x):
    w = -lax.log1p(-x * x)
    w_small = w - np.float32(2.5)
    p = jnp.full_like(x, np.float32(2.81022636e-08))
    for cc in (
        3.43273939e-07,
        -3.5233877e-06,
        -4.39150654e-06,
        0.00021858087,
        -0.00125372503,
        -0.00417768164,
        0.246640727,
        1.50140941,
    ):
        p = np.float32(cc) + p * w_small
    w_big = lax.sqrt(w) - np.float32(3.0)
    q = jnp.full_like(x, np.float32(-0.000200214257))
    for cc in (
        0.000100950558,
        0.00134934322,
        -0.00367342844,
        0.00573950773,
        -0.0076224613,
        0.00943887047,
        1.00167406,
        2.83297682,
    ):
        q = np.float32(cc) + q * w_big
    return jnp.where(w < np.float32(5.0), p, q) * x


def _normal(k):
    f = _u01(_bits(k))
    u = f * (np.float32(1.0) - _LO) + _LO
    u = jnp.maximum(_LO, u)
    return _SQRT2 * _erf_inv(u)


def _sample_loggamma(lane, alpha):
    z = jnp.zeros_like(lane)
    mk = (jnp.full_like(lane, _MK1), jnp.full_like(lane, _MK2))
    kk = _threefry(mk, z, lane)
    key0 = _threefry(kk, z, z)
    subkey = _threefry(kk, z, z + U32(1))

    boost = alpha >= np.float32(1.0)
    aprime = jnp.where(boost, alpha, alpha + np.float32(1.0))
    d = aprime - _THIRD
    c = _THIRD / lax.sqrt(d)
    f1 = np.float32(1.0)

    def percond(x2, v3, u):
        c1 = u >= f1 - _SQUEEZE * (x2 * x2)
        c2 = lax.log(u) >= x2 * np.float32(0.5) + d * (f1 - v3 + lax.log(v3))
        return c1 & c2

    def outer_cond(st):
        _, _, x2, v3, u = st
        return jnp.any(percond(x2, v3, u))

    def outer_body(st):
        k1_, k2_, x2, v3, u = st
        key = (k1_, k2_)
        m = percond(x2, v3, u)
        key_n = _threefry(key, z, z)
        x_key = _threefry(key, z, z + U32(1))
        u_key = _threefry(key, z, z + U32(2))

        def inner_cond(s):
            return jnp.any(s[3] <= np.float32(0.0))

        def inner_body(s):
            xk1, xk2, x, v = s
            mi = v <= np.float32(0.0)
            xk = (xk1, xk2)
            xk_n = _threefry(xk, z, z)
            sub = _threefry(xk, z, z + U32(1))
            xx = _normal(sub)
            vv = f1 + xx * c
            return (
                jnp.where(mi, xk_n[0], xk1),
                jnp.where(mi, xk_n[1], xk2),
                jnp.where(mi, xx, x),
                jnp.where(mi, vv, v),
            )

        _, _, x, v = lax.while_loop(
            inner_cond,
            inner_body,
            (x_key[0], x_key[1], jnp.zeros_like(alpha), jnp.full_like(alpha, -1.0)),
        )
        x2n = x * x
        v3n = (v * v) * v
        un = _u01(_bits(u_key))
        return (
            jnp.where(m, key_n[0], k1_),
            jnp.where(m, key_n[1], k2_),
            jnp.where(m, x2n, x2),
            jnp.where(m, v3n, v3),
            jnp.where(m, un, u),
        )

    init = (
        key0[0],
        key0[1],
        jnp.zeros_like(alpha),
        jnp.full_like(alpha, 1.0),
        jnp.full_like(alpha, 2.0),
    )
    _, _, _, v3, _ = lax.while_loop(outer_cond, outer_body, init)

    u_b = _u01(_bits(subkey))
    log_samples = lax.log1p(-u_b)
    log_boost = jnp.where(
        boost | (log_samples == np.float32(0.0)),
        np.float32(0.0),
        log_samples * (f1 / alpha),
    )
    return lax.log(d) + lax.log(v3) + log_boost


def _fused_body(prior_ref, rows_ref, alpha_ref, kl_ref, *, block_rows, kdim):
    bmw = block_rows
    wide = 2 * kdim
    conc = jnp.exp(rows_ref[...])

    c_iota = lax.broadcasted_iota(jnp.uint32, (bmw, wide), 1)
    mask_l = c_iota < U32(kdim)
    fzero = np.float32(0.0)
    fninf = np.float32(-np.inf)

    def half_sums(x):
        sl = jnp.sum(jnp.where(mask_l, x, fzero), axis=-1, keepdims=True)
        sr = jnp.sum(jnp.where(mask_l, fzero, x), axis=-1, keepdims=True)
        return sl, sr

    prior = prior_ref[...]
    qsum_l, qsum_r = half_sums(conc)
    psum = jnp.sum(jnp.where(mask_l[0:1], prior, fzero), axis=-1, keepdims=True)
    t1_l = _lgamma(qsum_l) - _lgamma(psum)
    t1_r = _lgamma(qsum_r) - _lgamma(psum)
    e2 = _lgamma(prior) - _lgamma(conc)
    t2_l, t2_r = half_sums(e2)
    qsum_sel = jnp.where(mask_l, qsum_l, qsum_r)
    e3 = (conc - prior) * (_digamma(conc) - _digamma(qsum_sel))
    t3_l, t3_r = half_sums(e3)
    kl_ref[...] = jnp.concatenate(
        [t1_l + t2_l + t3_l, t1_r + t2_r + t3_r], axis=1
    )

    r_iota = lax.broadcasted_iota(jnp.uint32, (bmw, wide), 0)
    base = lax.convert_element_type(pl.program_id(0), jnp.uint32) * U32(
        bmw * wide
    )
    lane = base + r_iota * U32(wide) + c_iota
    lg = _sample_loggamma(lane, conc)
    xmax_l = jnp.max(jnp.where(mask_l, lg, fninf), axis=-1, keepdims=True)
    xmax_r = jnp.max(jnp.where(mask_l, fninf, lg), axis=-1, keepdims=True)
    un = jnp.exp(lg - jnp.where(mask_l, xmax_l, xmax_r))
    s_l, s_r = half_sums(un)
    alpha_ref[...] = un / jnp.where(mask_l, s_l, s_r)


def _tc_fused(rows, prior_a, block_rows=32):
    B, K = rows.shape
    rows_w = rows.reshape(B // 2, 2 * K)
    prior_w = jnp.concatenate([prior_a, prior_a], axis=1)
    grid = ((B // 2) // block_rows,)
    alpha_w, kl_w = pl.pallas_call(
        functools.partial(_fused_body, block_rows=block_rows, kdim=K),
        grid=grid,
        compiler_params=pltpu.CompilerParams(
            dimension_semantics=("parallel",)
        ),
        in_specs=[
            pl.BlockSpec((1, 2 * K), lambda i: (0, 0)),
            pl.BlockSpec((block_rows, 2 * K), lambda i: (i, 0)),
        ],
        out_specs=[
            pl.BlockSpec((block_rows, 2 * K), lambda i: (i, 0)),
            pl.BlockSpec((block_rows, 2), lambda i: (i, 0)),
        ],
        out_shape=[
            jax.ShapeDtypeStruct((B // 2, 2 * K), jnp.float32),
            jax.ShapeDtypeStruct((B // 2, 2), jnp.float32),
        ],
    )(prior_w, rows_w)
    return alpha_w.reshape(B, K), kl_w.reshape(B, 1)


def kernel(idx, logc_weight, prior_a):
    rows = _sc_gather(logc_weight, idx)
    alpha, kl = _tc_fused(rows, prior_a)
    return (alpha, kl)


# speedup vs baseline: 1.1512x; 1.1512x over previous
import functools

import jax
import jax.numpy as jnp
import numpy as np
from jax import lax
from jax.experimental import pallas as pl
from jax.experimental.pallas import tpu as pltpu
from jax.experimental.pallas import tpu_sc as plsc

_IDX_CHUNK = 128


def _sc_gather(table, idx):
    B = idx.shape[0]
    D = table.shape[1]
    info = plsc.get_sparse_core_info()
    nw = info.num_cores * info.num_subcores
    b_per_w = B // nw
    n_chunks = b_per_w // _IDX_CHUNK
    mesh = plsc.VectorSubcoreMesh(core_axis_name="c", subcore_axis_name="s")

    @functools.partial(
        pl.kernel,
        mesh=mesh,
        compiler_params=pltpu.CompilerParams(use_tc_tiling_on_sc=False),
        out_type=jax.ShapeDtypeStruct((B, D), jnp.float32),
        scratch_types=[
            pltpu.VMEM((b_per_w,), jnp.int32),
            pltpu.VMEM((b_per_w, D), jnp.float32),
            pltpu.SemaphoreType.DMA,
        ],
    )
    def gather_kernel(table_hbm, idx_hbm, out_hbm, idx_v, rows_v, sem):
        wid = lax.axis_index("s") * info.num_cores + lax.axis_index("c")
        base = wid * b_per_w
        pltpu.sync_copy(idx_hbm.at[pl.ds(base, b_per_w)], idx_v)
        copies = []
        for j in range(n_chunks):
            copies.append(
                pltpu.async_copy(
                    table_hbm.at[idx_v.at[pl.ds(j * _IDX_CHUNK, _IDX_CHUNK)]],
                    rows_v.at[pl.ds(j * _IDX_CHUNK, _IDX_CHUNK)],
                    sem,
                )
            )
        for c in copies:
            c.wait()
        pltpu.sync_copy(rows_v, out_hbm.at[pl.ds(base, b_per_w)])

    return gather_kernel(table, idx.astype(jnp.int32))


_LANCZOS_G = 7.0
_LANCZOS_C = (
    0.99999999999980993,
    676.5203681218851,
    -1259.1392167224028,
    771.32342877765313,
    -176.61502916214059,
    12.507343278686905,
    -0.13857109526572012,
    9.9843695780195716e-6,
    1.5056327351493116e-7,
)
_HALF_LOG_2PI = 0.9189385332046727


def _lgamma(x):
    z = x - 1.0
    a = jnp.full_like(x, _LANCZOS_C[0])
    for i in range(1, 9):
        a = a + _LANCZOS_C[i] / (z + i)
    t = z + _LANCZOS_G + 0.5
    return _HALF_LOG_2PI + (z + 0.5) * jnp.log(t) - t + jnp.log(a)


def _digamma(x):
    shift = jnp.zeros_like(x)
    for k in range(6):
        shift = shift + 1.0 / (x + k)
    y = x + 6.0
    inv = 1.0 / y
    inv2 = inv * inv
    series = (
        jnp.log(y)
        - 0.5 * inv
        - inv2 * (1.0 / 12.0 - inv2 * (1.0 / 120.0 - inv2 / 252.0))
    )
    return series - shift


U32 = jnp.uint32
_R1 = (13, 15, 26, 6)
_R2 = (17, 29, 16, 24)
_C3 = np.uint32(0x1BD11BDA)
_LO = np.nextafter(np.float32(-1.0), np.float32(0.0))
_SQRT2 = np.array(np.sqrt(2), np.float32)
_THIRD = np.float32(1.0 / 3.0)
_SQUEEZE = np.float32(0.0331)
_MK1 = np.uint32(0)
_MK2 = np.uint32(12345)


def _tf_rounds(x0, x1, rots):
    for r in rots:
        x0 = x0 + x1
        x1 = (x1 << U32(r)) | (x1 >> U32(32 - r))
        x1 = x0 ^ x1
    return x0, x1


def _threefry(k, c0, c1):
    k1, k2 = k
    ks2 = k1 ^ k2 ^ _C3
    x0 = c0 + k1
    x1 = c1 + k2
    x0, x1 = _tf_rounds(x0, x1, _R1)
    x0 = x0 + k2
    x1 = x1 + ks2 + U32(1)
    x0, x1 = _tf_rounds(x0, x1, _R2)
    x0 = x0 + ks2
    x1 = x1 + k1 + U32(2)
    x0, x1 = _tf_rounds(x0, x1, _R1)
    x0 = x0 + k1
    x1 = x1 + k2 + U32(3)
    x0, x1 = _tf_rounds(x0, x1, _R2)
    x0 = x0 + k2
    x1 = x1 + ks2 + U32(4)
    x0, x1 = _tf_rounds(x0, x1, _R1)
    x0 = x0 + ks2
    x1 = x1 + k1 + U32(5)
    return x0, x1


def _u01(bits):
    fb = (bits >> U32(9)) | U32(0x3F800000)
    return lax.bitcast_convert_type(fb, jnp.float32) - np.float32(1.0)


def _bits(k):
    z = jnp.zeros_like(k[0])
    y0, y1 = _threefry(k, z, z)
    return y0 ^ y1


def _erf_inv(x):
    w = -lax.log1p(-x * x)
    w_small = w - np.float32(2.5)
    p = jnp.full_like(x, np.float32(2.81022636e-08))
    for cc in (
        3.43273939e-07,
        -3.5233877e-06,
        -4.39150654e-06,
        0.00021858087,
        -0.00125372503,
        -0.00417768164,
        0.246640727,
        1.50140941,
    ):
        p = np.float32(cc) + p * w_small
    w_big = lax.sqrt(w) - np.float32(3.0)
    q = jnp.full_like(x, np.float32(-0.000200214257))
    for cc in (
        0.000100950558,
        0.00134934322,
        -0.00367342844,
        0.00573950773,
        -0.0076224613,
        0.00943887047,
        1.00167406,
        2.83297682,
    ):
        q = np.float32(cc) + q * w_big
    return jnp.where(w < np.float32(5.0), p, q) * x


def _normal(k):
    f = _u01(_bits(k))
    u = f * (np.float32(1.0) - _LO) + _LO
    u = jnp.maximum(_LO, u)
    return _SQRT2 * _erf_inv(u)


def _sample_loggamma(lane, alpha):
    z = jnp.zeros_like(lane)
    mk = (jnp.full_like(lane, _MK1), jnp.full_like(lane, _MK2))
    kk = _threefry(mk, z, lane)
    key0 = _threefry(kk, z, z)
    subkey = _threefry(kk, z, z + U32(1))

    boost = alpha >= np.float32(1.0)
    aprime = jnp.where(boost, alpha, alpha + np.float32(1.0))
    d = aprime - _THIRD
    c = _THIRD / lax.sqrt(d)
    f1 = np.float32(1.0)

    def percond(x2, v3, u):
        c1 = u >= f1 - _SQUEEZE * (x2 * x2)
        c2 = lax.log(u) >= x2 * np.float32(0.5) + d * (f1 - v3 + lax.log(v3))
        return c1 & c2

    def outer_cond(st):
        _, _, x2, v3, u = st
        return jnp.any(percond(x2, v3, u))

    def outer_body(st):
        k1_, k2_, x2, v3, u = st
        key = (k1_, k2_)
        m = percond(x2, v3, u)
        key_n = _threefry(key, z, z)
        x_key = _threefry(key, z, z + U32(1))
        u_key = _threefry(key, z, z + U32(2))

        def inner_cond(s):
            return jnp.any(s[3] <= np.float32(0.0))

        def inner_body(s):
            xk1, xk2, x, v = s
            mi = v <= np.float32(0.0)
            xk = (xk1, xk2)
            xk_n = _threefry(xk, z, z)
            sub = _threefry(xk, z, z + U32(1))
            xx = _normal(sub)
            vv = f1 + xx * c
            return (
                jnp.where(mi, xk_n[0], xk1),
                jnp.where(mi, xk_n[1], xk2),
                jnp.where(mi, xx, x),
                jnp.where(mi, vv, v),
            )

        _, _, x, v = lax.while_loop(
            inner_cond,
            inner_body,
            (x_key[0], x_key[1], jnp.zeros_like(alpha), jnp.full_like(alpha, -1.0)),
        )
        x2n = x * x
        v3n = (v * v) * v
        un = _u01(_bits(u_key))
        return (
            jnp.where(m, key_n[0], k1_),
            jnp.where(m, key_n[1], k2_),
            jnp.where(m, x2n, x2),
            jnp.where(m, v3n, v3),
            jnp.where(m, un, u),
        )

    init = (
        key0[0],
        key0[1],
        jnp.zeros_like(alpha),
        jnp.full_like(alpha, 1.0),
        jnp.full_like(alpha, 2.0),
    )
    _, _, _, v3, _ = lax.while_loop(outer_cond, outer_body, init)

    u_b = _u01(_bits(subkey))
    log_samples = lax.log1p(-u_b)
    log_boost = jnp.where(
        boost | (log_samples == np.float32(0.0)),
        np.float32(0.0),
        log_samples * (f1 / alpha),
    )
    return lax.log(d) + lax.log(v3) + log_boost


def _fused_body(prior_ref, rows_ref, alpha_ref, kl_ref, *, block_rows, kdim):
    bmw = block_rows
    wide = 2 * kdim
    conc = jnp.exp(rows_ref[...])

    c_iota = lax.broadcasted_iota(jnp.uint32, (bmw, wide), 1)
    mask_l = c_iota < U32(kdim)
    fzero = np.float32(0.0)
    fninf = np.float32(-np.inf)

    def half_sums(x):
        sl = jnp.sum(jnp.where(mask_l, x, fzero), axis=-1, keepdims=True)
        sr = jnp.sum(jnp.where(mask_l, fzero, x), axis=-1, keepdims=True)
        return sl, sr

    prior = prior_ref[...]
    qsum_l, qsum_r = half_sums(conc)
    psum = jnp.sum(jnp.where(mask_l[0:1], prior, fzero), axis=-1, keepdims=True)
    t1_l = _lgamma(qsum_l) - _lgamma(psum)
    t1_r = _lgamma(qsum_r) - _lgamma(psum)
    e2 = _lgamma(prior) - _lgamma(conc)
    t2_l, t2_r = half_sums(e2)
    qsum_sel = jnp.where(mask_l, qsum_l, qsum_r)
    e3 = (conc - prior) * (_digamma(conc) - _digamma(qsum_sel))
    t3_l, t3_r = half_sums(e3)
    kl_ref[...] = jnp.concatenate(
        [t1_l + t2_l + t3_l, t1_r + t2_r + t3_r], axis=1
    )

    r_iota = lax.broadcasted_iota(jnp.uint32, (bmw, wide), 0)
    base = lax.convert_element_type(pl.program_id(0), jnp.uint32) * U32(
        bmw * wide
    )
    lane = base + r_iota * U32(wide) + c_iota
    lg = _sample_loggamma(lane, conc)
    xmax_l = jnp.max(jnp.where(mask_l, lg, fninf), axis=-1, keepdims=True)
    xmax_r = jnp.max(jnp.where(mask_l, fninf, lg), axis=-1, keepdims=True)
    un = jnp.exp(lg - jnp.where(mask_l, xmax_l, xmax_r))
    s_l, s_r = half_sums(un)
    alpha_ref[...] = un / jnp.where(mask_l, s_l, s_r)


def _tc_fused(rows, prior_a, block_rows=64):
    B, K = rows.shape
    rows_w = rows.reshape(B // 2, 2 * K)
    prior_w = jnp.concatenate([prior_a, prior_a], axis=1)
    grid = ((B // 2) // block_rows,)
    alpha_w, kl_w = pl.pallas_call(
        functools.partial(_fused_body, block_rows=block_rows, kdim=K),
        grid=grid,
        compiler_params=pltpu.CompilerParams(
            dimension_semantics=("parallel",)
        ),
        in_specs=[
            pl.BlockSpec((1, 2 * K), lambda i: (0, 0)),
            pl.BlockSpec((block_rows, 2 * K), lambda i: (i, 0)),
        ],
        out_specs=[
            pl.BlockSpec((block_rows, 2 * K), lambda i: (i, 0)),
            pl.BlockSpec((block_rows, 2), lambda i: (i, 0)),
        ],
        out_shape=[
            jax.ShapeDtypeStruct((B // 2, 2 * K), jnp.float32),
            jax.ShapeDtypeStruct((B // 2, 2), jnp.float32),
        ],
    )(prior_w, rows_w)
    return alpha_w.reshape(B, K), kl_w.reshape(B, 1)


def kernel(idx, logc_weight, prior_a):
    rows = _sc_gather(logc_weight, idx)
    alpha, kl = _tc_fused(rows, prior_a)
    return (alpha, kl)


# speedup vs baseline: 1.1778x; 1.0231x over previous
import functools

import jax
import jax.numpy as jnp
import numpy as np
from jax import lax
from jax.experimental import pallas as pl
from jax.experimental.pallas import tpu as pltpu
from jax.experimental.pallas import tpu_sc as plsc

_IDX_CHUNK = 128


def _sc_gather(table, idx):
    B = idx.shape[0]
    D = table.shape[1]
    info = plsc.get_sparse_core_info()
    nw = info.num_cores * info.num_subcores
    b_per_w = B // nw
    n_chunks = b_per_w // _IDX_CHUNK
    mesh = plsc.VectorSubcoreMesh(core_axis_name="c", subcore_axis_name="s")

    @functools.partial(
        pl.kernel,
        mesh=mesh,
        compiler_params=pltpu.CompilerParams(use_tc_tiling_on_sc=False),
        out_type=jax.ShapeDtypeStruct((B, D), jnp.float32),
        scratch_types=[
            pltpu.VMEM((b_per_w,), jnp.int32),
            pltpu.VMEM((b_per_w, D), jnp.float32),
            pltpu.SemaphoreType.DMA,
        ],
    )
    def gather_kernel(table_hbm, idx_hbm, out_hbm, idx_v, rows_v, sem):
        wid = lax.axis_index("s") * info.num_cores + lax.axis_index("c")
        base = wid * b_per_w
        pltpu.sync_copy(idx_hbm.at[pl.ds(base, b_per_w)], idx_v)
        copies = []
        for j in range(n_chunks):
            copies.append(
                pltpu.async_copy(
                    table_hbm.at[idx_v.at[pl.ds(j * _IDX_CHUNK, _IDX_CHUNK)]],
                    rows_v.at[pl.ds(j * _IDX_CHUNK, _IDX_CHUNK)],
                    sem,
                )
            )
        for c in copies:
            c.wait()
        pltpu.sync_copy(rows_v, out_hbm.at[pl.ds(base, b_per_w)])

    return gather_kernel(table, idx.astype(jnp.int32))


_LANCZOS_G = 7.0
_LANCZOS_C = (
    0.99999999999980993,
    676.5203681218851,
    -1259.1392167224028,
    771.32342877765313,
    -176.61502916214059,
    12.507343278686905,
    -0.13857109526572012,
    9.9843695780195716e-6,
    1.5056327351493116e-7,
)
_HALF_LOG_2PI = 0.9189385332046727


def _lgamma(x):
    z = x - 1.0
    a = jnp.full_like(x, _LANCZOS_C[0])
    for i in range(1, 9):
        a = a + _LANCZOS_C[i] / (z + i)
    t = z + _LANCZOS_G + 0.5
    return _HALF_LOG_2PI + (z + 0.5) * jnp.log(t) - t + jnp.log(a)


def _digamma(x):
    shift = jnp.zeros_like(x)
    for k in range(6):
        shift = shift + 1.0 / (x + k)
    y = x + 6.0
    inv = 1.0 / y
    inv2 = inv * inv
    series = (
        jnp.log(y)
        - 0.5 * inv
        - inv2 * (1.0 / 12.0 - inv2 * (1.0 / 120.0 - inv2 / 252.0))
    )
    return series - shift


U32 = jnp.uint32
_R1 = (13, 15, 26, 6)
_R2 = (17, 29, 16, 24)
_C3 = np.uint32(0x1BD11BDA)
_LO = np.nextafter(np.float32(-1.0), np.float32(0.0))
_SQRT2 = np.array(np.sqrt(2), np.float32)
_THIRD = np.float32(1.0 / 3.0)
_SQUEEZE = np.float32(0.0331)
_MK1 = np.uint32(0)
_MK2 = np.uint32(12345)


def _tf_rounds(x0, x1, rots):
    for r in rots:
        x0 = x0 + x1
        x1 = (x1 << U32(r)) | (x1 >> U32(32 - r))
        x1 = x0 ^ x1
    return x0, x1


def _threefry(k, c0, c1):
    k1, k2 = k
    ks2 = k1 ^ k2 ^ _C3
    x0 = c0 + k1
    x1 = c1 + k2
    x0, x1 = _tf_rounds(x0, x1, _R1)
    x0 = x0 + k2
    x1 = x1 + ks2 + U32(1)
    x0, x1 = _tf_rounds(x0, x1, _R2)
    x0 = x0 + ks2
    x1 = x1 + k1 + U32(2)
    x0, x1 = _tf_rounds(x0, x1, _R1)
    x0 = x0 + k1
    x1 = x1 + k2 + U32(3)
    x0, x1 = _tf_rounds(x0, x1, _R2)
    x0 = x0 + k2
    x1 = x1 + ks2 + U32(4)
    x0, x1 = _tf_rounds(x0, x1, _R1)
    x0 = x0 + ks2
    x1 = x1 + k1 + U32(5)
    return x0, x1


def _u01(bits):
    fb = (bits >> U32(9)) | U32(0x3F800000)
    return lax.bitcast_convert_type(fb, jnp.float32) - np.float32(1.0)


def _bits(k):
    z = jnp.zeros_like(k[0])
    y0, y1 = _threefry(k, z, z)
    return y0 ^ y1


def _erf_inv(x):
    w = -lax.log1p(-x * x)
    w_small = w - np.float32(2.5)
    p = jnp.full_like(x, np.float32(2.81022636e-08))
    for cc in (
        3.43273939e-07,
        -3.5233877e-06,
        -4.39150654e-06,
        0.00021858087,
        -0.00125372503,
        -0.00417768164,
        0.246640727,
        1.50140941,
    ):
        p = np.float32(cc) + p * w_small
    w_big = lax.sqrt(w) - np.float32(3.0)
    q = jnp.full_like(x, np.float32(-0.000200214257))
    for cc in (
        0.000100950558,
        0.00134934322,
        -0.00367342844,
        0.00573950773,
        -0.0076224613,
        0.00943887047,
        1.00167406,
        2.83297682,
    ):
        q = np.float32(cc) + q * w_big
    return jnp.where(w < np.float32(5.0), p, q) * x


def _normal(k):
    f = _u01(_bits(k))
    u = f * (np.float32(1.0) - _LO) + _LO
    u = jnp.maximum(_LO, u)
    return _SQRT2 * _erf_inv(u)


def _sample_loggamma(lane, alpha):
    z = jnp.zeros_like(lane)
    mk = (jnp.full_like(lane, _MK1), jnp.full_like(lane, _MK2))
    kk = _threefry(mk, z, lane)
    key0 = _threefry(kk, z, z)
    subkey = _threefry(kk, z, z + U32(1))

    boost = alpha >= np.float32(1.0)
    aprime = jnp.where(boost, alpha, alpha + np.float32(1.0))
    d = aprime - _THIRD
    c = _THIRD / lax.sqrt(d)
    f1 = np.float32(1.0)

    def percond(x2, v3, u):
        c1 = u >= f1 - _SQUEEZE * (x2 * x2)
        c2 = lax.log(u) >= x2 * np.float32(0.5) + d * (f1 - v3 + lax.log(v3))
        return c1 & c2

    def outer_cond(st):
        _, _, x2, v3, u = st
        return jnp.any(percond(x2, v3, u))

    def outer_body(st):
        k1_, k2_, x2, v3, u = st
        key = (k1_, k2_)
        m = percond(x2, v3, u)
        key_n = _threefry(key, z, z)
        x_key = _threefry(key, z, z + U32(1))
        u_key = _threefry(key, z, z + U32(2))

        def inner_cond(s):
            return jnp.any(s[3] <= np.float32(0.0))

        def inner_body(s):
            xk1, xk2, x, v = s
            mi = v <= np.float32(0.0)
            xk = (xk1, xk2)
            xk_n = _threefry(xk, z, z)
            sub = _threefry(xk, z, z + U32(1))
            xx = _normal(sub)
            vv = f1 + xx * c
            return (
                jnp.where(mi, xk_n[0], xk1),
                jnp.where(mi, xk_n[1], xk2),
                jnp.where(mi, xx, x),
                jnp.where(mi, vv, v),
            )

        _, _, x, v = lax.while_loop(
            inner_cond,
            inner_body,
            (x_key[0], x_key[1], jnp.zeros_like(alpha), jnp.full_like(alpha, -1.0)),
        )
        x2n = x * x
        v3n = (v * v) * v
        un = _u01(_bits(u_key))
        return (
            jnp.where(m, key_n[0], k1_),
            jnp.where(m, key_n[1], k2_),
            jnp.where(m, x2n, x2),
            jnp.where(m, v3n, v3),
            jnp.where(m, un, u),
        )

    key_n = _threefry(key0, z, z)
    x_key = _threefry(key0, z, z + U32(1))
    u_key = _threefry(key0, z, z + U32(2))
    sub = _threefry(x_key, z, z + U32(1))
    x1 = _normal(sub)
    v1 = f1 + x1 * c

    def inner_cond0(s):
        return jnp.any(s[3] <= np.float32(0.0))

    def inner_body0(s):
        xk1, xk2, x, v = s
        mi = v <= np.float32(0.0)
        xk = (xk1, xk2)
        xk_n = _threefry(xk, z, z)
        sub_i = _threefry(xk, z, z + U32(1))
        xx = _normal(sub_i)
        vv = f1 + xx * c
        return (
            jnp.where(mi, xk_n[0], xk1),
            jnp.where(mi, xk_n[1], xk2),
            jnp.where(mi, xx, x),
            jnp.where(mi, vv, v),
        )

    xk0 = _threefry(x_key, z, z)
    _, _, x1, v1 = lax.while_loop(
        inner_cond0, inner_body0, (xk0[0], xk0[1], x1, v1)
    )
    init = (
        key_n[0],
        key_n[1],
        x1 * x1,
        (v1 * v1) * v1,
        _u01(_bits(u_key)),
    )
    _, _, _, v3, _ = lax.while_loop(outer_cond, outer_body, init)

    u_b = _u01(_bits(subkey))
    log_samples = lax.log1p(-u_b)
    log_boost = jnp.where(
        boost | (log_samples == np.float32(0.0)),
        np.float32(0.0),
        log_samples * (f1 / alpha),
    )
    return lax.log(d) + lax.log(v3) + log_boost


def _fused_body(prior_ref, rows_ref, alpha_ref, kl_ref, *, block_rows, kdim):
    bmw = block_rows
    wide = 2 * kdim
    conc = jnp.exp(rows_ref[...])

    c_iota = lax.broadcasted_iota(jnp.uint32, (bmw, wide), 1)
    mask_l = c_iota < U32(kdim)
    fzero = np.float32(0.0)
    fninf = np.float32(-np.inf)

    def half_sums(x):
        sl = jnp.sum(jnp.where(mask_l, x, fzero), axis=-1, keepdims=True)
        sr = jnp.sum(jnp.where(mask_l, fzero, x), axis=-1, keepdims=True)
        return sl, sr

    prior = prior_ref[...]
    qsum_l, qsum_r = half_sums(conc)
    psum = jnp.sum(jnp.where(mask_l[0:1], prior, fzero), axis=-1, keepdims=True)
    t1_l = _lgamma(qsum_l) - _lgamma(psum)
    t1_r = _lgamma(qsum_r) - _lgamma(psum)
    e2 = _lgamma(prior) - _lgamma(conc)
    t2_l, t2_r = half_sums(e2)
    qsum_sel = jnp.where(mask_l, qsum_l, qsum_r)
    e3 = (conc - prior) * (_digamma(conc) - _digamma(qsum_sel))
    t3_l, t3_r = half_sums(e3)
    kl_ref[...] = jnp.concatenate(
        [t1_l + t2_l + t3_l, t1_r + t2_r + t3_r], axis=1
    )

    r_iota = lax.broadcasted_iota(jnp.uint32, (bmw, wide), 0)
    base = lax.convert_element_type(pl.program_id(0), jnp.uint32) * U32(
        bmw * wide
    )
    lane = base + r_iota * U32(wide) + c_iota
    lg = _sample_loggamma(lane, conc)
    xmax_l = jnp.max(jnp.where(mask_l, lg, fninf), axis=-1, keepdims=True)
    xmax_r = jnp.max(jnp.where(mask_l, fninf, lg), axis=-1, keepdims=True)
    un = jnp.exp(lg - jnp.where(mask_l, xmax_l, xmax_r))
    s_l, s_r = half_sums(un)
    alpha_ref[...] = un / jnp.where(mask_l, s_l, s_r)


def _tc_fused(rows, prior_a, block_rows=64):
    B, K = rows.shape
    rows_w = rows.reshape(B // 2, 2 * K)
    prior_w = jnp.concatenate([prior_a, prior_a], axis=1)
    grid = ((B // 2) // block_rows,)
    alpha_w, kl_w = pl.pallas_call(
        functools.partial(_fused_body, block_rows=block_rows, kdim=K),
        grid=grid,
        compiler_params=pltpu.CompilerParams(
            dimension_semantics=("parallel",)
        ),
        in_specs=[
            pl.BlockSpec((1, 2 * K), lambda i: (0, 0)),
            pl.BlockSpec((block_rows, 2 * K), lambda i: (i, 0)),
        ],
        out_specs=[
            pl.BlockSpec((block_rows, 2 * K), lambda i: (i, 0)),
            pl.BlockSpec((block_rows, 2), lambda i: (i, 0)),
        ],
        out_shape=[
            jax.ShapeDtypeStruct((B // 2, 2 * K), jnp.float32),
            jax.ShapeDtypeStruct((B // 2, 2), jnp.float32),
        ],
    )(prior_w, rows_w)
    return alpha_w.reshape(B, K), kl_w.reshape(B, 1)


def kernel(idx, logc_weight, prior_a):
    rows = _sc_gather(logc_weight, idx)
    alpha, kl = _tc_fused(rows, prior_a)
    return (alpha, kl)
